# baseline (speedup 1.0000x reference)
import functools

import jax
import jax.numpy as jnp
from jax import lax
from jax.experimental import pallas as pl
from jax.experimental.pallas import tpu as pltpu
from jax.experimental.pallas import tpu_sc as plsc

_H = 224
_W = 224
_HW = _H * _W
_N = 2
_C = 384
_NC = 2
_NS = 16
_NW = _NC * _NS
_CPW = (_N * _C) // _NW
_S = 1792
_NCHUNK = _HW // _S


def _prep_body(gx_ref, gy_ref, iw_ref):
    gx = gx_ref[...]
    gy = gy_ref[...]
    ix = ((gx + 1.0) * _W - 1.0) * 0.5
    iy = ((gy + 1.0) * _H - 1.0) * 0.5
    x0f = jnp.clip(jnp.floor(ix), -2.0, float(_W))
    y0f = jnp.clip(jnp.floor(iy), -2.0, float(_H))
    wx1 = ix - x0f
    wx0 = 1.0 - wx1
    wy1 = iy - y0f
    wy0 = 1.0 - wy1
    x0 = x0f.astype(jnp.int32)
    y0 = y0f.astype(jnp.int32)
    inx = ((x0 >= 0) & (x0 <= _W - 2)).astype(jnp.float32)
    iny = ((y0 >= 0) & (y0 <= _H - 2)).astype(jnp.float32)
    wl = wx0 * inx + wx1 * (x0 == -1)
    wr = wx1 * inx + wx0 * (x0 == _W - 1)
    wt = wy0 * iny + wy1 * (y0 == -1)
    wb = wy1 * iny + wy0 * (y0 == _H - 1)
    xb = jnp.clip(x0, 0, _W - 2)
    yb = jnp.clip(y0, 0, _H - 2)
    idx_f = lax.bitcast_convert_type(yb * _W + xb, jnp.float32)
    rows = [idx_f, wt * wl, wt * wr, wb * wl, wb * wr]
    iw_ref[...] = jnp.concatenate(
        [r.reshape(_N, _NCHUNK, _S) for r in rows], axis=-1
    )


def _prep(gx, gy):
    return pl.pallas_call(
        _prep_body,
        out_shape=jax.ShapeDtypeStruct((_N, _NCHUNK, 5 * _S), jnp.float32),
    )(gx, gy)


_mesh = plsc.VectorSubcoreMesh(core_axis_name="c", subcore_axis_name="s")


@functools.partial(
    pl.kernel,
    out_type=jax.ShapeDtypeStruct((_N * _C, _HW), jnp.float32),
    mesh=_mesh,
    compiler_params=pltpu.CompilerParams(needs_layout_passes=False),
    scratch_types=[
        pltpu.VMEM((_HW,), jnp.float32),
        pltpu.VMEM((_HW,), jnp.float32),
        (pltpu.VMEM((5 * _S,), jnp.float32),
         pltpu.VMEM((5 * _S,), jnp.float32)),
        pltpu.VMEM((2, 2, _S), jnp.float32),
        pltpu.SemaphoreType.DMA,
        (pltpu.SemaphoreType.DMA, pltpu.SemaphoreType.DMA),
        (pltpu.SemaphoreType.DMA, pltpu.SemaphoreType.DMA),
    ],
)
def _sc_sample(
    x_hbm, iw_hbm, out_hbm,
    plane0_v, plane1_v, iw_v, out_v, sem_pl, sem_iw, sem_out,
):
    wid = lax.axis_index("s") * _NC + lax.axis_index("c")
    base_plane = wid * _CPW
    b = base_plane // _C

    def iw_copy(c, buf):
        return pltpu.make_async_copy(iw_hbm.at[b, c], iw_v[buf], sem_iw[buf])

    def pair_body(pp, _):
        p0 = base_plane + 2 * pp
        cp0 = pltpu.async_copy(x_hbm.at[p0], plane0_v, sem_pl)
        cp1 = pltpu.async_copy(x_hbm.at[p0 + 1], plane1_v, sem_pl)
        iw_copy(0, 0).start()
        iw_copy(1, 1).start()
        cp0.wait()
        cp1.wait()

        def chunk2_body(cc, _):
            for buf in (0, 1):
                c = cc * 2 + buf
                off = c * _S
                iw_copy(c, buf).wait()

                @pl.when(c >= 2)
                def _():
                    pltpu.make_async_copy(
                        out_v.at[buf, 0], out_hbm.at[p0, pl.ds(off, _S)],
                        sem_out[buf],
                    ).wait()
                    pltpu.make_async_copy(
                        out_v.at[buf, 1], out_hbm.at[p0, pl.ds(off, _S)],
                        sem_out[buf],
                    ).wait()

                iwb = iw_v[buf]

                @plsc.parallel_loop(0, _S, step=16, unroll=8)
                def vec_body(o):
                    iv = plsc.bitcast(iwb[pl.ds(o, 16)], jnp.int32)
                    w00 = iwb[pl.ds(_S + o, 16)]
                    w01 = iwb[pl.ds(2 * _S + o, 16)]
                    w10 = iwb[pl.ds(3 * _S + o, 16)]
                    w11 = iwb[pl.ds(4 * _S + o, 16)]
                    iv1 = iv + 1
                    iv2 = iv + _W
                    iv3 = iv + (_W + 1)
                    for k, pk in ((0, plane0_v), (1, plane1_v)):
                        v00 = plsc.load_gather(pk, [iv])
                        v01 = plsc.load_gather(pk, [iv1])
                        v10 = plsc.load_gather(pk, [iv2])
                        v11 = plsc.load_gather(pk, [iv3])
                        out_v[buf, k, pl.ds(o, 16)] = (
                            v00 * w00 + v01 * w01 + v10 * w10 + v11 * w11
                        )

                @pl.when(c + 2 < _NCHUNK)
                def _():
                    iw_copy(c + 2, buf).start()

                pltpu.async_copy(
                    out_v.at[buf, 0], out_hbm.at[p0, pl.ds(off, _S)], sem_out[buf]
                )
                pltpu.async_copy(
                    out_v.at[buf, 1], out_hbm.at[p0 + 1, pl.ds(off, _S)],
                    sem_out[buf],
                )
            return 0

        lax.fori_loop(0, _NCHUNK // 2, chunk2_body, 0)
        for buf in (0, 1):
            pltpu.make_async_copy(
                out_v.at[buf, 0], out_hbm.at[p0, pl.ds(0, _S)], sem_out[buf]
            ).wait()
            pltpu.make_async_copy(
                out_v.at[buf, 1], out_hbm.at[p0, pl.ds(0, _S)], sem_out[buf]
            ).wait()
        return 0

    lax.fori_loop(0, _CPW // 2, pair_body, 0)


def kernel(x, grid):
    gx = grid[..., 0].reshape(_N, _HW)
    gy = grid[..., 1].reshape(_N, _HW)
    iw = _prep(gx, gy)
    out_flat = _sc_sample(x.reshape(_N * _C, _HW), iw)
    return out_flat.reshape(_N, _C, _H, _W)


# speedup vs baseline: 1.0049x; 1.0049x over previous
import functools

import jax
import jax.numpy as jnp
from jax import lax
from jax.experimental import pallas as pl
from jax.experimental.pallas import tpu as pltpu
from jax.experimental.pallas import tpu_sc as plsc

_H = 224
_W = 224
_HW = _H * _W
_N = 2
_C = 384
_NPP = _N * _C // 2
_NC = 2
_NS = 16
_NW = _NC * _NS
_PPW = _NPP // _NW
_S = 1024
_SR = _S // 128
_NCHUNK = _HW // _S
_CROWS = 5 * _SR


def _pack_body(x_ref, xp_ref):
    a = x_ref[0, 0]
    b = x_ref[0, 1]
    au = lax.bitcast_convert_type(a.astype(jnp.bfloat16), jnp.uint16)
    bu = lax.bitcast_convert_type(b.astype(jnp.bfloat16), jnp.uint16)
    word = (au.astype(jnp.uint32) << 16) | bu.astype(jnp.uint32)
    xp_ref[0] = lax.bitcast_convert_type(word, jnp.int32)


def _pack(x4):
    return pl.pallas_call(
        _pack_body,
        grid=(_NPP,),
        in_specs=[pl.BlockSpec((1, 2, 392, 128), lambda p: (p, 0, 0, 0))],
        out_specs=pl.BlockSpec((1, 392, 128), lambda p: (p, 0, 0)),
        out_shape=jax.ShapeDtypeStruct((_NPP, 392, 128), jnp.int32),
    )(x4)


def _prep_body(gx_ref, gy_ref, iw_ref):
    gx = gx_ref[...]
    gy = gy_ref[...]
    ix = ((gx + 1.0) * _W - 1.0) * 0.5
    iy = ((gy + 1.0) * _H - 1.0) * 0.5
    x0f = jnp.clip(jnp.floor(ix), -2.0, float(_W))
    y0f = jnp.clip(jnp.floor(iy), -2.0, float(_H))
    wx1 = ix - x0f
    wx0 = 1.0 - wx1
    wy1 = iy - y0f
    wy0 = 1.0 - wy1
    x0 = x0f.astype(jnp.int32)
    y0 = y0f.astype(jnp.int32)
    inx = ((x0 >= 0) & (x0 <= _W - 2)).astype(jnp.float32)
    iny = ((y0 >= 0) & (y0 <= _H - 2)).astype(jnp.float32)
    wl = wx0 * inx + wx1 * (x0 == -1)
    wr = wx1 * inx + wx0 * (x0 == _W - 1)
    wt = wy0 * iny + wy1 * (y0 == -1)
    wb = wy1 * iny + wy0 * (y0 == _H - 1)
    xb = jnp.clip(x0, 0, _W - 2)
    yb = jnp.clip(y0, 0, _H - 2)
    idx_f = lax.bitcast_convert_type(yb * _W + xb, jnp.float32)
    rows = [idx_f, wt * wl, wt * wr, wb * wl, wb * wr]
    stacked = jnp.stack(
        [r.reshape(_N, _NCHUNK, _SR, 128) for r in rows], axis=2
    )
    iw_ref[...] = stacked.reshape(_N, _NCHUNK * _CROWS, 128)


def _prep(gx, gy):
    return pl.pallas_call(
        _prep_body,
        out_shape=jax.ShapeDtypeStruct((_N, _NCHUNK * _CROWS, 128), jnp.float32),
    )(gx, gy)


_mesh = plsc.VectorSubcoreMesh(core_axis_name="c", subcore_axis_name="s")
_MASK_HI = -65536


@functools.partial(
    pl.kernel,
    out_type=jax.ShapeDtypeStruct((_N * _C, _HW), jnp.float32),
    mesh=_mesh,
    compiler_params=pltpu.CompilerParams(needs_layout_passes=False),
    scratch_types=[
        pltpu.VMEM((392, 128), jnp.int32),
        pltpu.VMEM((392, 128), jnp.int32),
        (pltpu.VMEM((_CROWS, 128), jnp.float32),
         pltpu.VMEM((_CROWS, 128), jnp.float32)),
        pltpu.VMEM((2, 4, _S), jnp.float32),
        pltpu.SemaphoreType.DMA,
        (pltpu.SemaphoreType.DMA, pltpu.SemaphoreType.DMA),
        (pltpu.SemaphoreType.DMA, pltpu.SemaphoreType.DMA),
    ],
)
def _sc_sample(
    xp_hbm, iw_hbm, out_hbm,
    pp0_v, pp1_v, iw_v, out_v, sem_pl, sem_iw, sem_out,
):
    wid = lax.axis_index("s") * _NC + lax.axis_index("c")
    base_pp = wid * _PPW
    b = base_pp // (_NPP // _N)

    def iw_copy(c, buf):
        return pltpu.make_async_copy(
            iw_hbm.at[b, pl.ds(c * _CROWS, _CROWS)], iw_v[buf], sem_iw[buf]
        )

    def pair_body(pp, _):
        q0 = base_pp + 2 * pp
        ch0 = 2 * q0
        cp0 = pltpu.async_copy(xp_hbm.at[q0], pp0_v, sem_pl)
        cp1 = pltpu.async_copy(xp_hbm.at[q0 + 1], pp1_v, sem_pl)
        iw_copy(0, 0).start()
        iw_copy(1, 1).start()
        cp0.wait()
        cp1.wait()

        def chunk_work(c, buf, first, last):
            off = c * _S
            iw_copy(c, buf).wait()

            def drain():
                for k in range(4):
                    pltpu.make_async_copy(
                        out_v.at[buf, k], out_hbm.at[ch0, pl.ds(off, _S)],
                        sem_out[buf],
                    ).wait()

            if first:
                pl.when(c >= 2)(drain)
            else:
                drain()

            iwb = iw_v[buf]

            for r in range(_SR):

                @plsc.parallel_loop(0, 128, step=16, unroll=4)
                def vec_body(co):
                    iv = plsc.bitcast(iwb[r, pl.ds(co, 16)], jnp.int32)
                    w00 = iwb[_SR + r, pl.ds(co, 16)]
                    w01 = iwb[2 * _SR + r, pl.ds(co, 16)]
                    w10 = iwb[3 * _SR + r, pl.ds(co, 16)]
                    w11 = iwb[4 * _SR + r, pl.ds(co, 16)]
                    iv1 = iv + 1
                    iv2 = iv + _W
                    iv3 = iv + (_W + 1)
                    taps = [(t >> 7, t & 127) for t in (iv, iv1, iv2, iv3)]
                    for k, ppv in ((0, pp0_v), (1, pp1_v)):
                        g00 = plsc.load_gather(ppv, list(taps[0]))
                        g01 = plsc.load_gather(ppv, list(taps[1]))
                        g10 = plsc.load_gather(ppv, list(taps[2]))
                        g11 = plsc.load_gather(ppv, list(taps[3]))
                        acc_a = (
                            plsc.bitcast(g00 & _MASK_HI, jnp.float32) * w00
                            + plsc.bitcast(g01 & _MASK_HI, jnp.float32) * w01
                            + plsc.bitcast(g10 & _MASK_HI, jnp.float32) * w10
                            + plsc.bitcast(g11 & _MASK_HI, jnp.float32) * w11
                        )
                        acc_b = (
                            plsc.bitcast(g00 << 16, jnp.float32) * w00
                            + plsc.bitcast(g01 << 16, jnp.float32) * w01
                            + plsc.bitcast(g10 << 16, jnp.float32) * w10
                            + plsc.bitcast(g11 << 16, jnp.float32) * w11
                        )
                        o = r * 128 + co
                        out_v[buf, 2 * k, pl.ds(o, 16)] = acc_a
                        out_v[buf, 2 * k + 1, pl.ds(o, 16)] = acc_b

            if not last:
                pl.when(c + 2 < _NCHUNK)(lambda: iw_copy(c + 2, buf).start())

            for k in range(4):
                pltpu.async_copy(
                    out_v.at[buf, k], out_hbm.at[ch0 + k, pl.ds(off, _S)],
                    sem_out[buf],
                )

        def chunk2_body(cc, _):
            for buf in (0, 1):
                chunk_work(cc * 2 + buf, buf, first=True, last=False)
            return 0

        lax.fori_loop(0, _NCHUNK // 2, chunk2_body, 0)
        chunk_work(_NCHUNK - 1, 0, first=False, last=True)
        for buf in (0, 1):
            for k in range(4):
                pltpu.make_async_copy(
                    out_v.at[buf, k], out_hbm.at[ch0, pl.ds(0, _S)], sem_out[buf]
                ).wait()
        return 0

    lax.fori_loop(0, _PPW // 2, pair_body, 0)


def kernel(x, grid):
    xp = _pack(x.reshape(_NPP, 2, 392, 128))
    gx = grid[..., 0].reshape(_N, _HW)
    gy = grid[..., 1].reshape(_N, _HW)
    iw = _prep(gx, gy)
    out_flat = _sc_sample(xp, iw)
    return out_flat.reshape(_N, _C, _H, _W)


# speedup vs baseline: 1.2544x; 1.2483x over previous
import functools

import jax
import jax.numpy as jnp
from jax import lax
from jax.experimental import pallas as pl
from jax.experimental.pallas import tpu as pltpu
from jax.experimental.pallas import tpu_sc as plsc

_H = 224
_W = 224
_HW = _H * _W
_N = 2
_C = 384
_CH = _C // 2
_NC = 2
_NS = 16
_NW = _NC * _NS
_PPW = _N * _CH // _NW
_RC = 2
_S = _RC * _W
_NCHUNK = _H // _RC
_CROWS = 16
_MASK_HI = -65536


def _pack_body(x_ref, xp_ref):
    for j in range(4):
        a = x_ref[0, 2 * j]
        b = x_ref[0, 2 * j + 1]
        au = lax.bitcast_convert_type(a.astype(jnp.bfloat16), jnp.uint16)
        bu = lax.bitcast_convert_type(b.astype(jnp.bfloat16), jnp.uint16)
        word = (au.astype(jnp.uint32) << 16) | bu.astype(jnp.uint32)
        w = lax.bitcast_convert_type(word, jnp.int32)
        xp_ref[0, j] = jnp.pad(w, ((0, 0), (0, 32)))


def _pack(x):
    return pl.pallas_call(
        _pack_body,
        grid=(_N, _C // 8),
        in_specs=[pl.BlockSpec((1, 8, _H, _W), lambda n, p: (n, p, 0, 0))],
        out_specs=pl.BlockSpec((1, 4, _H, 256), lambda n, p: (n, p, 0, 0)),
        out_shape=jax.ShapeDtypeStruct((_N, _CH, _H, 256), jnp.int32),
    )(x)


def _prep_body(gx_ref, gy_ref, iw_ref):
    gx = gx_ref[...]
    gy = gy_ref[...]
    ix = ((gx + 1.0) * _W - 1.0) * 0.5
    iy = ((gy + 1.0) * _H - 1.0) * 0.5
    x0f = jnp.clip(jnp.floor(ix), -2.0, float(_W))
    y0f = jnp.clip(jnp.floor(iy), -2.0, float(_H))
    wx1 = ix - x0f
    wx0 = 1.0 - wx1
    wy1 = iy - y0f
    wy0 = 1.0 - wy1
    x0 = x0f.astype(jnp.int32)
    y0 = y0f.astype(jnp.int32)
    inx = ((x0 >= 0) & (x0 <= _W - 2)).astype(jnp.float32)
    iny = ((y0 >= 0) & (y0 <= _H - 2)).astype(jnp.float32)
    wl = wx0 * inx + wx1 * (x0 == -1)
    wr = wx1 * inx + wx0 * (x0 == _W - 1)
    wt = wy0 * iny + wy1 * (y0 == -1)
    wb = wy1 * iny + wy0 * (y0 == _H - 1)
    xb = jnp.clip(x0, 0, _W - 2)
    yb = jnp.clip(y0, 0, _H - 2)
    iv_f = lax.bitcast_convert_type((yb << 8) | xb, jnp.float32)
    rows = [iv_f, wt * wl, wt * wr, wb * wl, wb * wr]
    stacked = jnp.stack(
        [r.reshape(_N, _NCHUNK, _RC, _W) for r in rows], axis=2
    ).reshape(_N, _NCHUNK, 5 * _RC, _W)
    padded = jnp.pad(
        stacked, ((0, 0), (0, 0), (0, _CROWS - 5 * _RC), (0, 256 - _W))
    )
    iw_ref[...] = padded.reshape(_N, _NCHUNK * _CROWS, 256)


def _prep(gx, gy):
    return pl.pallas_call(
        _prep_body,
        out_shape=jax.ShapeDtypeStruct((_N, _NCHUNK * _CROWS, 256), jnp.float32),
    )(gx, gy)


_mesh = plsc.VectorSubcoreMesh(core_axis_name="c", subcore_axis_name="s")


@functools.partial(
    pl.kernel,
    out_type=jax.ShapeDtypeStruct((_N, _C, _H, _W), jnp.float32),
    mesh=_mesh,
    compiler_params=pltpu.CompilerParams(needs_layout_passes=False),
    scratch_types=[
        pltpu.VMEM((_H, 256), jnp.int32),
        pltpu.VMEM((_H, 256), jnp.int32),
        (pltpu.VMEM((_CROWS, 256), jnp.float32),
         pltpu.VMEM((_CROWS, 256), jnp.float32)),
        pltpu.VMEM((2, 4, _RC, _W), jnp.float32),
        pltpu.SemaphoreType.DMA,
        (pltpu.SemaphoreType.DMA, pltpu.SemaphoreType.DMA),
        (pltpu.SemaphoreType.DMA, pltpu.SemaphoreType.DMA),
    ],
)
def _sc_sample(
    xp_hbm, iw_hbm, out_hbm,
    pp0_v, pp1_v, iw_v, out_v, sem_pl, sem_iw, sem_out,
):
    wid = lax.axis_index("s") * _NC + lax.axis_index("c")
    base_pp = wid * _PPW
    b = base_pp // _CH
    base_q = base_pp - b * _CH

    def iw_copy(c, buf):
        return pltpu.make_async_copy(
            iw_hbm.at[b, pl.ds(c * _CROWS, _CROWS)], iw_v[buf], sem_iw[buf]
        )

    def out_copy(buf, k, ch, c):
        return pltpu.make_async_copy(
            out_v.at[buf, k],
            out_hbm.at[b, ch + k, pl.ds(c * _RC, _RC)],
            sem_out[buf],
        )

    def pair_body(pp, _):
        q0 = base_q + 2 * pp
        ch0 = 2 * q0
        cp0 = pltpu.async_copy(xp_hbm.at[b, q0], pp0_v, sem_pl)
        cp1 = pltpu.async_copy(xp_hbm.at[b, q0 + 1], pp1_v, sem_pl)
        iw_copy(0, 0).start()
        iw_copy(1, 1).start()
        cp0.wait()
        cp1.wait()

        def chunk2_body(cc, _):
            for buf in (0, 1):
                c = cc * 2 + buf
                iw_copy(c, buf).wait()

                @pl.when(c >= 2)
                def _():
                    for k in range(4):
                        out_copy(buf, k, ch0, c).wait()

                iwb = iw_v[buf]

                for r in range(_RC):

                    @plsc.parallel_loop(0, _W, step=16, unroll=4)
                    def vec_body(x0):
                        iv = plsc.bitcast(iwb[r, pl.ds(x0, 16)], jnp.int32)
                        w00 = iwb[_RC + r, pl.ds(x0, 16)]
                        w01 = iwb[2 * _RC + r, pl.ds(x0, 16)]
                        w10 = iwb[3 * _RC + r, pl.ds(x0, 16)]
                        w11 = iwb[4 * _RC + r, pl.ds(x0, 16)]
                        yv = iv >> 8
                        xv = iv & 255
                        y1 = yv + 1
                        x1 = xv + 1
                        for k, ppv in ((0, pp0_v), (1, pp1_v)):
                            g00 = plsc.load_gather(ppv, [yv, xv])
                            g01 = plsc.load_gather(ppv, [yv, x1])
                            g10 = plsc.load_gather(ppv, [y1, xv])
                            g11 = plsc.load_gather(ppv, [y1, x1])
                            acc_a = (
                                plsc.bitcast(g00 & _MASK_HI, jnp.float32) * w00
                                + plsc.bitcast(g01 & _MASK_HI, jnp.float32) * w01
                                + plsc.bitcast(g10 & _MASK_HI, jnp.float32) * w10
                                + plsc.bitcast(g11 & _MASK_HI, jnp.float32) * w11
                            )
                            acc_b = (
                                plsc.bitcast(g00 << 16, jnp.float32) * w00
                                + plsc.bitcast(g01 << 16, jnp.float32) * w01
                                + plsc.bitcast(g10 << 16, jnp.float32) * w10
                                + plsc.bitcast(g11 << 16, jnp.float32) * w11
                            )
                            out_v[buf, 2 * k, r, pl.ds(x0, 16)] = acc_a
                            out_v[buf, 2 * k + 1, r, pl.ds(x0, 16)] = acc_b

                @pl.when(c + 2 < _NCHUNK)
                def _():
                    iw_copy(c + 2, buf).start()

                for k in range(4):
                    out_copy(buf, k, ch0, c).start()
            return 0

        lax.fori_loop(0, _NCHUNK // 2, chunk2_body, 0)
        for buf in (0, 1):
            for k in range(4):
                out_copy(buf, k, ch0, 0).wait()
        return 0

    lax.fori_loop(0, _PPW // 2, pair_body, 0)


def kernel(x, grid):
    xp = _pack(x)
    gx = grid[..., 0].reshape(_N, _HW)
    gy = grid[..., 1].reshape(_N, _HW)
    iw = _prep(gx, gy)
    return _sc_sample(xp, iw)


# speedup vs baseline: 1.3635x; 1.0869x over previous
import functools

import jax
import jax.numpy as jnp
from jax import lax
from jax.experimental import pallas as pl
from jax.experimental.pallas import tpu as pltpu
from jax.experimental.pallas import tpu_sc as plsc

_H = 224
_W = 224
_HW = _H * _W
_N = 2
_C = 384
_CH = _C // 2
_NC = 2
_NS = 16
_NW = _NC * _NS
_PPW = _N * _CH // _NW
_RC = 2
_S = _RC * _W
_NCHUNK = _H // _RC
_CROWS = 16
_MASK_HI = -65536


def _pack_body(x_ref, xp_ref):
    for j in range(4):
        a = x_ref[2 * j]
        b = x_ref[2 * j + 1]
        au = lax.bitcast_convert_type(a.astype(jnp.bfloat16), jnp.uint16)
        bu = lax.bitcast_convert_type(b.astype(jnp.bfloat16), jnp.uint16)
        word = (au.astype(jnp.uint32) << 16) | bu.astype(jnp.uint32)
        w = lax.bitcast_convert_type(word, jnp.int32)
        xp_ref[j] = jnp.pad(w, ((0, 0), (0, 32)))


def _pack(x):
    return pl.pallas_call(
        _pack_body,
        grid=(_N * _C // 8,),
        in_specs=[pl.BlockSpec((8, _H, _W), lambda p: (p, 0, 0))],
        out_specs=pl.BlockSpec((4, _H, 256), lambda p: (p, 0, 0)),
        out_shape=jax.ShapeDtypeStruct((_N * _CH, _H, 256), jnp.int32),
    )(x.reshape(_N * _C, _H, _W))


def _prep_body(gx_ref, gy_ref, iw_ref):
    gx = gx_ref[...]
    gy = gy_ref[...]
    ix = ((gx + 1.0) * _W - 1.0) * 0.5
    iy = ((gy + 1.0) * _H - 1.0) * 0.5
    x0f = jnp.clip(jnp.floor(ix), -2.0, float(_W))
    y0f = jnp.clip(jnp.floor(iy), -2.0, float(_H))
    wx1 = ix - x0f
    wx0 = 1.0 - wx1
    wy1 = iy - y0f
    wy0 = 1.0 - wy1
    x0 = x0f.astype(jnp.int32)
    y0 = y0f.astype(jnp.int32)
    inx = ((x0 >= 0) & (x0 <= _W - 2)).astype(jnp.float32)
    iny = ((y0 >= 0) & (y0 <= _H - 2)).astype(jnp.float32)
    wl = wx0 * inx + wx1 * (x0 == -1)
    wr = wx1 * inx + wx0 * (x0 == _W - 1)
    wt = wy0 * iny + wy1 * (y0 == -1)
    wb = wy1 * iny + wy0 * (y0 == _H - 1)
    xb = jnp.clip(x0, 0, _W - 2)
    yb = jnp.clip(y0, 0, _H - 2)
    iv_f = lax.bitcast_convert_type((yb << 8) | xb, jnp.float32)
    rows = [iv_f, wt * wl, wt * wr, wb * wl, wb * wr]
    stacked = jnp.stack(
        [r.reshape(_N, _NCHUNK, _RC, _W) for r in rows], axis=2
    ).reshape(_N, _NCHUNK, 5 * _RC, _W)
    padded = jnp.pad(
        stacked, ((0, 0), (0, 0), (0, _CROWS - 5 * _RC), (0, 256 - _W))
    )
    iw_ref[...] = padded.reshape(_N, _NCHUNK * _CROWS, 256)


def _prep(gx, gy):
    return pl.pallas_call(
        _prep_body,
        out_shape=jax.ShapeDtypeStruct((_N, _NCHUNK * _CROWS, 256), jnp.float32),
    )(gx, gy)


_mesh = plsc.VectorSubcoreMesh(core_axis_name="c", subcore_axis_name="s")


@functools.partial(
    pl.kernel,
    out_type=jax.ShapeDtypeStruct((_N, _C, _H, _W), jnp.float32),
    mesh=_mesh,
    compiler_params=pltpu.CompilerParams(needs_layout_passes=False),
    scratch_types=[
        pltpu.VMEM((_H, 256), jnp.int32),
        pltpu.VMEM((_H, 256), jnp.int32),
        (pltpu.VMEM((_CROWS, 256), jnp.float32),
         pltpu.VMEM((_CROWS, 256), jnp.float32)),
        pltpu.VMEM((2, 4, _RC, _W), jnp.float32),
        pltpu.SemaphoreType.DMA,
        (pltpu.SemaphoreType.DMA, pltpu.SemaphoreType.DMA),
        (pltpu.SemaphoreType.DMA, pltpu.SemaphoreType.DMA),
    ],
)
def _sc_sample(
    xp_hbm, iw_hbm, out_hbm,
    pp0_v, pp1_v, iw_v, out_v, sem_pl, sem_iw, sem_out,
):
    wid = lax.axis_index("s") * _NC + lax.axis_index("c")
    base_pp = wid * _PPW
    b = base_pp // _CH
    base_q = base_pp - b * _CH

    def iw_copy(c, buf):
        return pltpu.make_async_copy(
            iw_hbm.at[b, pl.ds(c * _CROWS, _CROWS)], iw_v[buf], sem_iw[buf]
        )

    def out_copy(buf, ch, c):
        return pltpu.make_async_copy(
            out_v.at[buf],
            out_hbm.at[b, pl.ds(ch, 4), pl.ds(c * _RC, _RC)],
            sem_out[buf],
        )

    def pair_body(pp, _):
        q0 = base_pp + 2 * pp
        ch0 = 2 * (base_q + 2 * pp)
        cp0 = pltpu.async_copy(xp_hbm.at[q0], pp0_v, sem_pl)
        cp1 = pltpu.async_copy(xp_hbm.at[q0 + 1], pp1_v, sem_pl)
        iw_copy(0, 0).start()
        iw_copy(1, 1).start()
        cp0.wait()
        cp1.wait()

        def chunk2_body(cc, _):
            for buf in (0, 1):
                c = cc * 2 + buf
                iw_copy(c, buf).wait()

                @pl.when(c >= 2)
                def _():
                    out_copy(buf, ch0, c).wait()

                iwb = iw_v[buf]

                @plsc.parallel_loop(0, _W, step=16, unroll=2)
                def vec_body(x0):
                    for r in range(_RC):
                        iv = plsc.bitcast(iwb[r, pl.ds(x0, 16)], jnp.int32)
                        w00 = iwb[_RC + r, pl.ds(x0, 16)]
                        w01 = iwb[2 * _RC + r, pl.ds(x0, 16)]
                        w10 = iwb[3 * _RC + r, pl.ds(x0, 16)]
                        w11 = iwb[4 * _RC + r, pl.ds(x0, 16)]
                        yv = iv >> 8
                        xv = iv & 255
                        y1 = yv + 1
                        x1 = xv + 1
                        for k, ppv in ((0, pp0_v), (1, pp1_v)):
                            g00 = plsc.load_gather(ppv, [yv, xv])
                            g01 = plsc.load_gather(ppv, [yv, x1])
                            g10 = plsc.load_gather(ppv, [y1, xv])
                            g11 = plsc.load_gather(ppv, [y1, x1])
                            acc_a = (
                                plsc.bitcast(g00 & _MASK_HI, jnp.float32) * w00
                                + plsc.bitcast(g01 & _MASK_HI, jnp.float32) * w01
                                + plsc.bitcast(g10 & _MASK_HI, jnp.float32) * w10
                                + plsc.bitcast(g11 & _MASK_HI, jnp.float32) * w11
                            )
                            acc_b = (
                                plsc.bitcast(g00 << 16, jnp.float32) * w00
                                + plsc.bitcast(g01 << 16, jnp.float32) * w01
                                + plsc.bitcast(g10 << 16, jnp.float32) * w10
                                + plsc.bitcast(g11 << 16, jnp.float32) * w11
                            )
                            out_v[buf, 2 * k, r, pl.ds(x0, 16)] = acc_a
                            out_v[buf, 2 * k + 1, r, pl.ds(x0, 16)] = acc_b

                @pl.when(c + 2 < _NCHUNK)
                def _():
                    iw_copy(c + 2, buf).start()

                out_copy(buf, ch0, c).start()
            return 0

        lax.fori_loop(0, _NCHUNK // 2, chunk2_body, 0)
        for buf in (0, 1):
            out_copy(buf, ch0, 0).wait()
        return 0

    lax.fori_loop(0, _PPW // 2, pair_body, 0)


def kernel(x, grid):
    xp = _pack(x)
    gx = grid[..., 0].reshape(_N, _HW)
    gy = grid[..., 1].reshape(_N, _HW)
    iw = _prep(gx, gy)
    return _sc_sample(xp, iw)
